# initial kernel scaffold (unmeasured)
import numpy as np

import jax
import jax.numpy as jnp
from jax import lax
from jax.experimental import pallas as pl
from jax.experimental.pallas import tpu as pltpu

N_DEV = 8
B, SQ, D = 2, 512, 1024
T = B * SQ
HL, DH = 8, 128
CH = T // N_DEV
SCALE = 0.08838834764831843


def _rope_tables():
    inv = 1.0 / (10000.0 ** (np.arange(0, DH, 2) / DH))
    pos = np.arange(SQ)[:, None] * inv[None, :]
    ck, sk = np.cos(pos), np.sin(pos)
    cos2 = np.concatenate([ck, ck], axis=1)
    sin2 = np.concatenate([-sk, sk], axis=1)
    cos_t = np.concatenate([cos2, cos2], axis=0)
    sin_t = np.concatenate([sin2, sin2], axis=0)
    return cos_t.astype(np.float32), sin_t.astype(np.float32)


_COS, _SIN = _rope_tables()


def _deinterleave_cols(w):
    return w.reshape(D, HL, DH // 2, 2).transpose(0, 1, 3, 2).reshape(D, HL * DH)


def _body(x_ref, wq_ref, wk_ref, wv_ref, wo_ref, cos_ref, sin_ref, out_ref,
          qb, kb, vb, ctx, part, sbuf, rs_rbuf, ag_rbuf,
          rs_ssem, rs_rsem, ag_ssem, ag_rsem):
    me = lax.axis_index("i")
    left = (me + N_DEV - 1) % N_DEV
    right = (me + 1) % N_DEV

    barrier_sem = pltpu.get_barrier_semaphore()
    for nbr in (left, right):
        pl.semaphore_signal(barrier_sem, inc=1, device_id=(nbr,),
                            device_id_type=pl.DeviceIdType.MESH)
    pl.semaphore_wait(barrier_sem, 2)

    qb[:, :] = jnp.dot(x_ref[:, :], wq_ref[:, :], preferred_element_type=jnp.float32)
    kb[:, :] = jnp.dot(x_ref[:, :], wk_ref[:, :], preferred_element_type=jnp.float32)
    vb[:, :] = jnp.dot(x_ref[:, :], wv_ref[:, :], preferred_element_type=jnp.float32)

    cos = cos_ref[:, :]
    sin = sin_ref[:, :]
    for h in range(HL):
        cs = slice(h * DH, (h + 1) * DH)
        q = qb[:, cs]
        qb[:, cs] = q * cos + pltpu.roll(q, 64, 1) * sin
        k = kb[:, cs]
        kb[:, cs] = k * cos + pltpu.roll(k, 64, 1) * sin

    for h in range(HL):
        cs = slice(h * DH, (h + 1) * DH)
        for b in range(B):
            rs = slice(b * SQ, (b + 1) * SQ)
            q = qb[rs, cs]
            k = kb[rs, cs]
            v = vb[rs, cs]
            s = lax.dot_general(q, k, (((1,), (1,)), ((), ())),
                                preferred_element_type=jnp.float32) * SCALE
            m = jnp.max(s, axis=1, keepdims=True)
            e = jnp.exp(s - m)
            den = jnp.sum(e, axis=1, keepdims=True)
            ctx[rs, cs] = jnp.dot(e, v, preferred_element_type=jnp.float32) / den

    part[:, :] = jnp.dot(ctx[:, :], wo_ref[:, :], preferred_element_type=jnp.float32)

    def chunk(i):
        return pl.ds(i * CH, CH)

    sbuf[0, :, :] = part[chunk(me), :]
    for s in range(N_DEV - 1):
        rdma = pltpu.make_async_remote_copy(
            src_ref=sbuf.at[s],
            dst_ref=rs_rbuf.at[s],
            send_sem=rs_ssem.at[s],
            recv_sem=rs_rsem.at[s],
            device_id=(right,),
            device_id_type=pl.DeviceIdType.MESH,
        )
        rdma.start()
        rdma.wait()
        recv_idx = (me + 2 * N_DEV - s - 1) % N_DEV
        sbuf[s + 1, :, :] = rs_rbuf[s, :, :] + part[chunk(recv_idx), :]
    out_ref[chunk((me + 1) % N_DEV), :] = sbuf[N_DEV - 1, :, :]

    for t in range(N_DEV - 1):
        src = sbuf.at[N_DEV - 1] if t == 0 else ag_rbuf.at[t - 1]
        rdma = pltpu.make_async_remote_copy(
            src_ref=src,
            dst_ref=ag_rbuf.at[t],
            send_sem=ag_ssem.at[t],
            recv_sem=ag_rsem.at[t],
            device_id=(right,),
            device_id_type=pl.DeviceIdType.MESH,
        )
        rdma.start()
        rdma.wait()
        out_idx = (me + 2 * N_DEV - t) % N_DEV
        out_ref[chunk(out_idx), :] = ag_rbuf[t, :, :]


def kernel(x, Wq, Wk, Wv, Wo):
    x2 = x.reshape(T, D)
    wq = _deinterleave_cols(Wq)
    wk = _deinterleave_cols(Wk)
    cos_t = jnp.asarray(_COS)
    sin_t = jnp.asarray(_SIN)

    out = pl.pallas_call(
        _body,
        out_shape=jax.ShapeDtypeStruct((T, D), jnp.float32),
        in_specs=[pl.BlockSpec(memory_space=pltpu.VMEM)] * 7,
        out_specs=pl.BlockSpec(memory_space=pltpu.VMEM),
        scratch_shapes=[
            pltpu.VMEM((T, HL * DH), jnp.float32),
            pltpu.VMEM((T, HL * DH), jnp.float32),
            pltpu.VMEM((T, HL * DH), jnp.float32),
            pltpu.VMEM((T, HL * DH), jnp.float32),
            pltpu.VMEM((T, D), jnp.float32),
            pltpu.VMEM((N_DEV, CH, D), jnp.float32),
            pltpu.VMEM((N_DEV - 1, CH, D), jnp.float32),
            pltpu.VMEM((N_DEV - 1, CH, D), jnp.float32),
            pltpu.SemaphoreType.DMA((N_DEV - 1,)),
            pltpu.SemaphoreType.DMA((N_DEV - 1,)),
            pltpu.SemaphoreType.DMA((N_DEV - 1,)),
            pltpu.SemaphoreType.DMA((N_DEV - 1,)),
        ],
        compiler_params=pltpu.CompilerParams(collective_id=0),
    )(x2, wq, wk, Wv, Wo, cos_t, sin_t)
    return out.reshape(B, SQ, D)


# baseline (device time: 161115 ns/iter reference)
import numpy as np

import jax
import jax.numpy as jnp
from jax import lax
from jax.experimental import pallas as pl
from jax.experimental.pallas import tpu as pltpu

N_DEV = 8
B, SQ, D = 2, 512, 1024
T = B * SQ
HL, DH = 8, 128
CH = T // N_DEV
SCALE = 0.08838834764831843


def _rope_tables():
    inv = 1.0 / (10000.0 ** (np.arange(0, DH, 2) / DH))
    pos = np.arange(SQ)[:, None] * inv[None, :]
    ck, sk = np.cos(pos), np.sin(pos)
    cos2 = np.concatenate([ck, ck], axis=1)
    sin2 = np.concatenate([-sk, sk], axis=1)
    cos_t = np.concatenate([cos2, cos2], axis=0)
    sin_t = np.concatenate([sin2, sin2], axis=0)
    return cos_t.astype(np.float32), sin_t.astype(np.float32)


_COS, _SIN = _rope_tables()


def _deinterleave_cols(w):
    return w.reshape(D, HL, DH // 2, 2).transpose(0, 1, 3, 2).reshape(D, HL * DH)


def _body(x_ref, wq_ref, wk_ref, wv_ref, wo_ref, cos_ref, sin_ref, out_ref,
          qb, kb, vb, sbuf, rs_rbuf, ag_rbuf,
          rs_ssem, rs_rsem, ag_ssem, ag_rsem):
    me = lax.axis_index("i")
    left = (me + N_DEV - 1) % N_DEV
    right = (me + 1) % N_DEV

    barrier_sem = pltpu.get_barrier_semaphore()
    for nbr in (left, right):
        pl.semaphore_signal(barrier_sem, inc=1, device_id=(nbr,),
                            device_id_type=pl.DeviceIdType.MESH)
    pl.semaphore_wait(barrier_sem, 2)

    qb[:, :] = jnp.dot(x_ref[:, :], wq_ref[:, :], preferred_element_type=jnp.float32)
    kb[:, :] = jnp.dot(x_ref[:, :], wk_ref[:, :], preferred_element_type=jnp.float32)
    vb[:, :] = jnp.dot(x_ref[:, :], wv_ref[:, :], preferred_element_type=jnp.float32)

    cos = cos_ref[:, :]
    sin = sin_ref[:, :]
    for h in range(HL):
        cs = slice(h * DH, (h + 1) * DH)
        q = qb[:, cs]
        qb[:, cs] = q * cos + pltpu.roll(q, 64, 1) * sin
        k = kb[:, cs]
        kb[:, cs] = k * cos + pltpu.roll(k, 64, 1) * sin

    for h in range(HL):
        cs = slice(h * DH, (h + 1) * DH)
        for b in range(B):
            rs = slice(b * SQ, (b + 1) * SQ)
            q = qb[rs, cs]
            k = kb[rs, cs]
            v = vb[rs, cs]
            s = lax.dot_general(q, k, (((1,), (1,)), ((), ())),
                                preferred_element_type=jnp.float32) * SCALE
            m = jnp.max(s, axis=1, keepdims=True)
            e = jnp.exp(s - m)
            den = jnp.sum(e, axis=1, keepdims=True)
            vb[rs, cs] = jnp.dot(e, v, preferred_element_type=jnp.float32) / den

    out_ref[:, :] = jnp.dot(vb[:, :], wo_ref[:, :], preferred_element_type=jnp.float32)
    part = out_ref

    def chunk(i):
        return pl.ds(i * CH, CH)

    sbuf[0, :, :] = part[chunk(me), :]
    for s in range(N_DEV - 1):
        rdma = pltpu.make_async_remote_copy(
            src_ref=sbuf.at[s],
            dst_ref=rs_rbuf.at[s],
            send_sem=rs_ssem.at[s],
            recv_sem=rs_rsem.at[s],
            device_id=(right,),
            device_id_type=pl.DeviceIdType.MESH,
        )
        rdma.start()
        rdma.wait()
        recv_idx = (me + 2 * N_DEV - s - 1) % N_DEV
        sbuf[s + 1, :, :] = rs_rbuf[s, :, :] + part[chunk(recv_idx), :]
    out_ref[chunk((me + 1) % N_DEV), :] = sbuf[N_DEV - 1, :, :]

    for t in range(N_DEV - 1):
        src = sbuf.at[N_DEV - 1] if t == 0 else ag_rbuf.at[t - 1]
        rdma = pltpu.make_async_remote_copy(
            src_ref=src,
            dst_ref=ag_rbuf.at[t],
            send_sem=ag_ssem.at[t],
            recv_sem=ag_rsem.at[t],
            device_id=(right,),
            device_id_type=pl.DeviceIdType.MESH,
        )
        rdma.start()
        rdma.wait()
        out_idx = (me + 2 * N_DEV - t) % N_DEV
        out_ref[chunk(out_idx), :] = ag_rbuf[t, :, :]


def kernel(x, Wq, Wk, Wv, Wo):
    x2 = x.reshape(T, D)
    wq = _deinterleave_cols(Wq)
    wk = _deinterleave_cols(Wk)
    cos_t = jnp.asarray(_COS)
    sin_t = jnp.asarray(_SIN)

    out = pl.pallas_call(
        _body,
        out_shape=jax.ShapeDtypeStruct((T, D), jnp.float32),
        in_specs=[pl.BlockSpec(memory_space=pltpu.VMEM)] * 7,
        out_specs=pl.BlockSpec(memory_space=pltpu.VMEM),
        scratch_shapes=[
            pltpu.VMEM((T, HL * DH), jnp.float32),
            pltpu.VMEM((T, HL * DH), jnp.float32),
            pltpu.VMEM((T, HL * DH), jnp.float32),
            pltpu.VMEM((N_DEV, CH, D), jnp.float32),
            pltpu.VMEM((N_DEV - 1, CH, D), jnp.float32),
            pltpu.VMEM((N_DEV - 1, CH, D), jnp.float32),
            pltpu.SemaphoreType.DMA((N_DEV - 1,)),
            pltpu.SemaphoreType.DMA((N_DEV - 1,)),
            pltpu.SemaphoreType.DMA((N_DEV - 1,)),
            pltpu.SemaphoreType.DMA((N_DEV - 1,)),
        ],
        compiler_params=pltpu.CompilerParams(
            collective_id=0, vmem_limit_bytes=100 * 1024 * 1024
        ),
    )(x2, wq, wk, Wv, Wo, cos_t, sin_t)
    return out.reshape(B, SQ, D)


# device time: 99186 ns/iter; 1.6244x vs baseline; 1.6244x over previous
import numpy as np

import jax
import jax.numpy as jnp
from jax import lax
from jax.experimental import pallas as pl
from jax.experimental.pallas import tpu as pltpu

N_DEV = 8
B, SQ, D = 2, 512, 1024
T = B * SQ
HL, DH = 8, 128
CH = T // N_DEV
SCALE = 0.08838834764831843
WIRE = jnp.bfloat16


def _rope_tables():
    inv = 1.0 / (10000.0 ** (np.arange(0, DH, 2) / DH))
    pos = np.arange(SQ)[:, None] * inv[None, :]
    ck, sk = np.cos(pos), np.sin(pos)
    cos2 = np.concatenate([ck, ck], axis=1)
    sin2 = np.concatenate([-sk, sk], axis=1)
    cos_t = np.concatenate([cos2, cos2], axis=0)
    sin_t = np.concatenate([sin2, sin2], axis=0)
    return cos_t.astype(np.float32), sin_t.astype(np.float32)


_COS, _SIN = _rope_tables()


def _deinterleave_cols(w):
    return w.reshape(D, HL, DH // 2, 2).transpose(0, 1, 3, 2).reshape(D, HL * DH)


def _vid(d):
    return d ^ ((d >> 1) & 1)


def _body(x_ref, wq_ref, wk_ref, wv_ref, wo_ref, cos_ref, sin_ref, out_ref,
          qb, kb, vb, sb1, sb2, sb3, rb1, rb2, rb3,
          ab1, ab2, ab3, gr1, gr2, gr3, ssem, rsem):
    me = lax.axis_index("i")
    vm = _vid(me)
    prt = [_vid(vm ^ 1), _vid(vm ^ 2), _vid(vm ^ 4)]

    barrier_sem = pltpu.get_barrier_semaphore()
    for nbr in prt:
        pl.semaphore_signal(barrier_sem, inc=1, device_id=(nbr,),
                            device_id_type=pl.DeviceIdType.MESH)
    pl.semaphore_wait(barrier_sem, 3)

    def exchange(k, src, dst, partner):
        return pltpu.make_async_remote_copy(
            src_ref=src, dst_ref=dst,
            send_sem=ssem.at[k], recv_sem=rsem.at[k],
            device_id=(partner,), device_id_type=pl.DeviceIdType.MESH,
        )

    qb[:, :] = jnp.dot(x_ref[:, :], wq_ref[:, :], preferred_element_type=jnp.float32)
    kb[:, :] = jnp.dot(x_ref[:, :], wk_ref[:, :], preferred_element_type=jnp.float32)
    vb[:, :] = jnp.dot(x_ref[:, :], wv_ref[:, :], preferred_element_type=jnp.float32)
    cos = cos_ref[:, :]
    sin = sin_ref[:, :]
    for h in range(HL):
        cs = slice(h * DH, (h + 1) * DH)
        q = qb[:, cs]
        qb[:, cs] = q * cos + pltpu.roll(q, 64, 1) * sin
        k = kb[:, cs]
        kb[:, cs] = k * cos + pltpu.roll(k, 64, 1) * sin

    def attn_and_partial(lo):
        rows = pl.ds(lo, SQ)
        for h in range(HL):
            cs = slice(h * DH, (h + 1) * DH)
            q = qb[rows, cs]
            k = kb[rows, cs]
            v = vb[rows, cs]
            s = lax.dot_general(q, k, (((1,), (1,)), ((), ())),
                                preferred_element_type=jnp.float32) * SCALE
            m = jnp.max(s, axis=1, keepdims=True)
            e = jnp.exp(s - m)
            den = jnp.sum(e, axis=1, keepdims=True)
            vb[rows, cs] = jnp.dot(e, v, preferred_element_type=jnp.float32) / den
        out_ref[rows, :] = jnp.dot(vb[rows, :], wo_ref[:, :],
                                   preferred_element_type=jnp.float32)

    bit2 = (vm >> 2) & 1
    bit1 = (vm >> 1) & 1
    bit0 = vm & 1
    lo_send1 = (1 - bit2) * 512
    lo_keep1 = bit2 * 512

    attn_and_partial(lo_send1)
    sb1[:, :] = out_ref[pl.ds(lo_send1, 512), :].astype(WIRE)
    ex1 = exchange(0, sb1, rb1, prt[2])
    ex1.start()
    attn_and_partial(lo_keep1)
    ex1.wait()
    out_ref[pl.ds(lo_keep1, 512), :] = (
        out_ref[pl.ds(lo_keep1, 512), :] + rb1[:, :].astype(jnp.float32)
    )

    lo_send2 = lo_keep1 + (1 - bit1) * 256
    lo_keep2 = lo_keep1 + bit1 * 256
    sb2[:, :] = out_ref[pl.ds(lo_send2, 256), :].astype(WIRE)
    ex2 = exchange(1, sb2, rb2, prt[1])
    ex2.start()
    ex2.wait()
    out_ref[pl.ds(lo_keep2, 256), :] = (
        out_ref[pl.ds(lo_keep2, 256), :] + rb2[:, :].astype(jnp.float32)
    )

    lo_send3 = lo_keep2 + (1 - bit0) * 128
    lo_keep3 = lo_keep2 + bit0 * 128
    sb3[:, :] = out_ref[pl.ds(lo_send3, 128), :].astype(WIRE)
    ex3 = exchange(2, sb3, rb3, prt[0])
    ex3.start()
    ex3.wait()
    out_ref[pl.ds(lo_keep3, 128), :] = (
        out_ref[pl.ds(lo_keep3, 128), :] + rb3[:, :].astype(jnp.float32)
    )

    lo = lo_keep3
    ab1[:, :] = out_ref[pl.ds(lo, 128), :].astype(WIRE)
    ex4 = exchange(3, ab1, gr1, prt[0])
    ex4.start()
    ex4.wait()
    out_ref[pl.ds(lo + (1 - 2 * bit0) * 128, 128), :] = gr1[:, :].astype(jnp.float32)
    lo = lo - bit0 * 128

    ab2[:, :] = out_ref[pl.ds(lo, 256), :].astype(WIRE)
    ex5 = exchange(4, ab2, gr2, prt[1])
    ex5.start()
    ex5.wait()
    out_ref[pl.ds(lo + (1 - 2 * bit1) * 256, 256), :] = gr2[:, :].astype(jnp.float32)
    lo = lo - bit1 * 256

    ab3[:, :] = out_ref[pl.ds(lo, 512), :].astype(WIRE)
    ex6 = exchange(5, ab3, gr3, prt[2])
    ex6.start()
    ex6.wait()
    out_ref[pl.ds(lo + (1 - 2 * bit2) * 512, 512), :] = gr3[:, :].astype(jnp.float32)


def kernel(x, Wq, Wk, Wv, Wo):
    x2 = x.reshape(T, D)
    wq = _deinterleave_cols(Wq)
    wk = _deinterleave_cols(Wk)
    cos_t = jnp.asarray(_COS)
    sin_t = jnp.asarray(_SIN)

    out = pl.pallas_call(
        _body,
        out_shape=jax.ShapeDtypeStruct((T, D), jnp.float32),
        in_specs=[pl.BlockSpec(memory_space=pltpu.VMEM)] * 7,
        out_specs=pl.BlockSpec(memory_space=pltpu.VMEM),
        scratch_shapes=[
            pltpu.VMEM((T, HL * DH), jnp.float32),
            pltpu.VMEM((T, HL * DH), jnp.float32),
            pltpu.VMEM((T, HL * DH), jnp.float32),
            pltpu.VMEM((512, D), WIRE),
            pltpu.VMEM((256, D), WIRE),
            pltpu.VMEM((128, D), WIRE),
            pltpu.VMEM((512, D), WIRE),
            pltpu.VMEM((256, D), WIRE),
            pltpu.VMEM((128, D), WIRE),
            pltpu.VMEM((128, D), WIRE),
            pltpu.VMEM((256, D), WIRE),
            pltpu.VMEM((512, D), WIRE),
            pltpu.VMEM((128, D), WIRE),
            pltpu.VMEM((256, D), WIRE),
            pltpu.VMEM((512, D), WIRE),
            pltpu.SemaphoreType.DMA((6,)),
            pltpu.SemaphoreType.DMA((6,)),
        ],
        compiler_params=pltpu.CompilerParams(
            collective_id=0, vmem_limit_bytes=100 * 1024 * 1024
        ),
    )(x2, wq, wk, Wv, Wo, cos_t, sin_t)
    return out.reshape(B, SQ, D)
